# baseline (speedup 1.0000x reference)
import jax
import jax.numpy as jnp
from jax import lax
from jax.experimental import pallas as pl
from jax.experimental.pallas import tpu as pltpu
from jax.experimental.pallas import tpu_sc as plsc

N_ROWS = 16384
N_FEATS = 512
OUT_F = 64

NC = 2
NS = 16
NW = NC * NS
ROWS_PER_W = N_ROWS // NW
BLK = 64
NBLK = ROWS_PER_W // BLK
OBLK = 2 * BLK


def _sc_body(x_hbm, cols_hbm, out_hbm,
             cols_v, xa, xb, oa, ob, sxa, sxb, soa, sob):
    wid = lax.axis_index("s") * NC + lax.axis_index("c")
    row_base = wid * ROWS_PER_W
    pltpu.sync_copy(cols_hbm, cols_v)

    riota = lax.broadcasted_iota(jnp.int32, (16,), 0)
    zeros = jnp.zeros((16,), jnp.int32)

    x_bufs = (xa, xb)
    o_bufs = (oa, ob)
    x_sems = (sxa, sxb)
    o_sems = (soa, sob)

    def issue_x(b):
        start = row_base + b * BLK
        return pltpu.async_copy(
            x_hbm.at[pl.ds(start, BLK)], x_bufs[b % 2], x_sems[b % 2])

    def issue_o(k):
        start = row_base + k * OBLK
        return pltpu.async_copy(
            o_bufs[k % 2], out_hbm.at[:, pl.ds(start, OBLK)], o_sems[k % 2])

    o_descs = {}
    d = issue_x(0)
    for b in range(NBLK):
        d_next = issue_x(b + 1) if b + 1 < NBLK else None
        d.wait()
        k = b // 2
        if b % 2 == 0 and k >= 2:
            o_descs[k - 2].wait()
        x_v = x_bufs[b % 2]
        o_v = o_bufs[k % 2]
        half = (b % 2) * BLK

        def do_col(j, _, x_v=x_v, o_v=o_v, half=half):
            cspl = plsc.load_gather(cols_v, [zeros + j])
            for u in range(BLK // 16):
                ridx = riota + (u * 16)
                g = plsc.load_gather(x_v, [ridx, cspl])
                o_v[j, pl.ds(half + u * 16, 16)] = g
            return 0

        lax.fori_loop(0, OUT_F, do_col, 0)
        if b % 2 == 1:
            o_descs[k] = issue_o(k)
        d = d_next

    o_descs[NBLK // 2 - 2].wait()
    o_descs[NBLK // 2 - 1].wait()


@jax.jit
def _sc_gather(x, cols):
    mesh = plsc.VectorSubcoreMesh(core_axis_name="c", subcore_axis_name="s")
    return pl.kernel(
        _sc_body,
        out_type=jax.ShapeDtypeStruct((OUT_F, N_ROWS), jnp.float32),
        mesh=mesh,
        scratch_types=[
            pltpu.VMEM((OUT_F,), jnp.int32),
            pltpu.VMEM((BLK, N_FEATS), jnp.float32),
            pltpu.VMEM((BLK, N_FEATS), jnp.float32),
            pltpu.VMEM((OUT_F, OBLK), jnp.float32),
            pltpu.VMEM((OUT_F, OBLK), jnp.float32),
            pltpu.SemaphoreType.DMA,
            pltpu.SemaphoreType.DMA,
            pltpu.SemaphoreType.DMA,
            pltpu.SemaphoreType.DMA,
        ],
        compiler_params=pltpu.CompilerParams(needs_layout_passes=False),
    )(x, cols)


def kernel(x, columns):
    out_t = _sc_gather(x, columns.astype(jnp.int32))
    return out_t.T


# speedup vs baseline: 1.3977x; 1.3977x over previous
import jax
import jax.numpy as jnp
from jax import lax
from jax.experimental import pallas as pl
from jax.experimental.pallas import tpu as pltpu
from jax.experimental.pallas import tpu_sc as plsc

N_ROWS = 16384
N_FEATS = 512
OUT_F = 64

SPLIT = 4096

NC = 2
NS = 16
NW = NC * NS
ROWS_PER_W = SPLIT // NW
BLK = 64
NBLK = ROWS_PER_W // BLK
OBLK = 2 * BLK
NOB = NBLK // 2

TC_BLOCK = 1024
TC_ROWS = N_ROWS - SPLIT


def _sc_body(x_hbm, cols_hbm, out_hbm,
             cols_v, xa, xb, oa, ob, sxa, sxb, soa, sob):
    wid = lax.axis_index("s") * NC + lax.axis_index("c")
    row_base = wid * ROWS_PER_W
    pltpu.sync_copy(cols_hbm, cols_v)

    riota = lax.broadcasted_iota(jnp.int32, (16,), 0)
    zeros = jnp.zeros((16,), jnp.int32)

    x_bufs = (xa, xb)
    o_bufs = (oa, ob)
    x_sems = (sxa, sxb)
    o_sems = (soa, sob)

    def issue_x(b):
        start = row_base + b * BLK
        return pltpu.async_copy(
            x_hbm.at[pl.ds(start, BLK)], x_bufs[b % 2], x_sems[b % 2])

    def issue_o(k):
        start = row_base + k * OBLK
        return pltpu.async_copy(
            o_bufs[k % 2], out_hbm.at[:, pl.ds(start, OBLK)], o_sems[k % 2])

    o_descs = {}
    d = issue_x(0)
    for b in range(NBLK):
        d_next = issue_x(b + 1) if b + 1 < NBLK else None
        d.wait()
        k = b // 2
        if b % 2 == 0 and k >= 2:
            o_descs[k - 2].wait()
        x_v = x_bufs[b % 2]
        o_v = o_bufs[k % 2]
        half = (b % 2) * BLK

        def do_col(j, _, x_v=x_v, o_v=o_v, half=half):
            cspl = plsc.load_gather(cols_v, [zeros + j])
            for u in range(BLK // 16):
                ridx = riota + (u * 16)
                g = plsc.load_gather(x_v, [ridx, cspl])
                o_v[j, pl.ds(half + u * 16, 16)] = g
            return 0

        lax.fori_loop(0, OUT_F, do_col, 0)
        if b % 2 == 1:
            o_descs[k] = issue_o(k)
        d = d_next

    if NOB >= 2:
        o_descs[NOB - 2].wait()
    o_descs[NOB - 1].wait()


def _sc_gather(x, cols):
    mesh = plsc.VectorSubcoreMesh(core_axis_name="c", subcore_axis_name="s")
    return pl.kernel(
        _sc_body,
        out_type=jax.ShapeDtypeStruct((OUT_F, SPLIT), jnp.float32),
        mesh=mesh,
        scratch_types=[
            pltpu.VMEM((OUT_F,), jnp.int32),
            pltpu.VMEM((BLK, N_FEATS), jnp.float32),
            pltpu.VMEM((BLK, N_FEATS), jnp.float32),
            pltpu.VMEM((OUT_F, OBLK), jnp.float32),
            pltpu.VMEM((OUT_F, OBLK), jnp.float32),
            pltpu.SemaphoreType.DMA,
            pltpu.SemaphoreType.DMA,
            pltpu.SemaphoreType.DMA,
            pltpu.SemaphoreType.DMA,
        ],
        compiler_params=pltpu.CompilerParams(needs_layout_passes=False),
    )(x, cols)


def _tc_body(cols_ref, x_ref, o_ref):
    cols = cols_ref[0:1, :]
    iota_c = lax.broadcasted_iota(jnp.int32, (N_FEATS, OUT_F), 0)
    onehot = (iota_c == cols).astype(jnp.float32)
    o_ref[...] = lax.dot_general(
        onehot, x_ref[...], (((0,), (1,)), ((), ())),
        preferred_element_type=jnp.float32)


def _tc_gather(x, cols2d):
    return pl.pallas_call(
        _tc_body,
        grid=(TC_ROWS // TC_BLOCK,),
        in_specs=[
            pl.BlockSpec((8, OUT_F), lambda i: (0, 0)),
            pl.BlockSpec((TC_BLOCK, N_FEATS), lambda i: (SPLIT // TC_BLOCK + i, 0)),
        ],
        out_specs=pl.BlockSpec((OUT_F, TC_BLOCK), lambda i: (0, i)),
        out_shape=jax.ShapeDtypeStruct((OUT_F, TC_ROWS), jnp.float32),
    )(cols2d, x)


@jax.jit
def _gather(x, cols):
    cols2d = jnp.broadcast_to(cols[None, :], (8, OUT_F))
    out_sc = _sc_gather(x, cols)
    out_tc = _tc_gather(x, cols2d)
    out_t = jnp.concatenate([out_sc, out_tc], axis=1)
    return out_t.T


def kernel(x, columns):
    return _gather(x, columns.astype(jnp.int32))
